# baseline (device time: 28631 ns/iter reference)
import jax
import jax.numpy as jnp
from jax import lax
from jax.experimental import pallas as pl
from jax.experimental.pallas import tpu as pltpu

N_DEV = 4
CAP = 384


def kernel(x, router_W, route_idx, expert_W, shared_W):
    n_tok, d_model = x.shape
    e_local = expert_W.shape[0]
    d_ff = shared_W.shape[1]
    n_exp = router_W.shape[1]
    half = d_ff // 2

    def body_a(x_ref, rw_ref, idx_ref, ew_ref, sw_ref,
               part_ref, blkr_ref, blkl_ref, pos_ref,
               comm_r, comm_l, send_r, recv_r, send_l, recv_l):
        my_pos = lax.axis_index("i")
        left = (my_pos + N_DEV - 1) % N_DEV
        right = (my_pos + 1) % N_DEV

        barrier_sem = pltpu.get_barrier_semaphore()
        for nbr in (left, right):
            pl.semaphore_signal(
                barrier_sem, inc=1,
                device_id=(nbr,), device_id_type=pl.DeviceIdType.MESH,
            )
        pl.semaphore_wait(barrier_sem, 2)

        xf = x_ref[:, :]

        scores = jnp.dot(xf, rw_ref[:, :], preferred_element_type=jnp.float32)
        s_max = jnp.max(scores, axis=-1, keepdims=True)
        p = jnp.exp(scores - s_max)
        probs = p / jnp.sum(p, axis=-1, keepdims=True)

        route = idx_ref[:, :]
        eidx = lax.broadcasted_iota(jnp.int32, (n_tok, n_exp), 1)
        onehot = (eidx == route).astype(jnp.float32)
        p_tok = jnp.sum(probs * onehot, axis=-1, keepdims=True)

        route_chip = route // e_local
        cidx = lax.broadcasted_iota(jnp.int32, (n_tok, N_DEV), 1)
        chip_masks = (cidx == route_chip).astype(jnp.float32)
        r_iota = lax.broadcasted_iota(jnp.int32, (n_tok, n_tok), 0)
        c_iota = lax.broadcasted_iota(jnp.int32, (n_tok, n_tok), 1)
        l_strict = (c_iota < r_iota).astype(jnp.bfloat16)
        pos_f = jnp.dot(
            l_strict, chip_masks.astype(jnp.bfloat16),
            preferred_element_type=jnp.float32,
        )
        pos_ref[:, :] = pos_f

        slot_iota = lax.broadcasted_iota(jnp.int32, (n_tok, CAP), 1)
        mask_me = (route_chip == my_pos).astype(jnp.float32)
        pos_me = jnp.sum(
            pos_f * (cidx == my_pos).astype(jnp.float32),
            axis=1, keepdims=True,
        ).astype(jnp.int32)
        s_me = (
            (slot_iota == pos_me).astype(jnp.float32) * mask_me
        ).astype(jnp.bfloat16)

        x_scaled = (xf * p_tok).astype(jnp.bfloat16)
        x_c = lax.dot_general(
            s_me, x_scaled,
            dimension_numbers=(((0,), (0,)), ((), ())),
            preferred_element_type=jnp.float32,
        ).astype(jnp.bfloat16)
        le_iota = lax.broadcasted_iota(jnp.int32, (n_tok, e_local), 1)
        local_onehot = (
            le_iota == route - my_pos * e_local
        ).astype(jnp.bfloat16)
        masks_c = lax.dot_general(
            s_me, local_onehot,
            dimension_numbers=(((0,), (0,)), ((), ())),
            preferred_element_type=jnp.float32,
        ).astype(jnp.bfloat16)

        y_c = jnp.zeros((CAP, d_ff), jnp.float32)
        for e in range(e_local):
            xs = x_c * masks_c[:, e:e + 1]
            y_c = y_c + jnp.dot(
                xs, ew_ref[e, :, :].astype(jnp.bfloat16),
                preferred_element_type=jnp.float32,
            )
        y_bf = y_c.astype(jnp.bfloat16)

        comm_r[0, :, :] = y_bf[:, :half]
        comm_l[0, :, :] = y_bf[:, half:]
        blkr_ref[N_DEV - 1, :, :] = y_bf[:, :half]
        blkl_ref[N_DEV - 1, :, :] = y_bf[:, half:]

        def make_hop(h):
            send_slot = h % 2
            recv_slot = (h + 1) % 2
            rdma_r = pltpu.make_async_remote_copy(
                src_ref=comm_r.at[send_slot],
                dst_ref=comm_r.at[recv_slot],
                send_sem=send_r.at[h],
                recv_sem=recv_r.at[h],
                device_id=(right,),
                device_id_type=pl.DeviceIdType.MESH,
            )
            rdma_l = pltpu.make_async_remote_copy(
                src_ref=comm_l.at[send_slot],
                dst_ref=comm_l.at[recv_slot],
                send_sem=send_l.at[h],
                recv_sem=recv_l.at[h],
                device_id=(left,),
                device_id_type=pl.DeviceIdType.MESH,
            )
            rdma_r.start()
            rdma_l.start()
            return rdma_r, rdma_l

        hop = make_hop(0)
        part_ref[:, :] = jnp.dot(
            xf.astype(jnp.bfloat16), sw_ref[:, :].astype(jnp.bfloat16),
            preferred_element_type=jnp.float32,
        )

        for h in range(N_DEV - 1):
            hop[0].wait()
            hop[1].wait()
            recv_slot = (h + 1) % 2
            if h + 1 < N_DEV - 1:
                hop = make_hop(h + 1)
            blkr_ref[h, :, :] = comm_r[recv_slot, :, :]
            blkl_ref[h, :, :] = comm_l[recv_slot, :, :]

    def body_b(idx_ref, pos_ref, part_ref, blkr_ref, blkl_ref, out_ref):
        my_pos = lax.axis_index("i")

        route = idx_ref[:, :]
        owner = route // e_local
        cidx = lax.broadcasted_iota(jnp.int32, (n_tok, N_DEV), 1)
        pos_f = pos_ref[:, :]
        pos_own = jnp.sum(
            pos_f * (cidx == owner).astype(jnp.float32),
            axis=1, keepdims=True,
        ).astype(jnp.int32)

        stack_iota = lax.broadcasted_iota(
            jnp.int32, (n_tok, N_DEV * CAP), 1
        )
        hop_r = (my_pos - owner - 1 + 2 * N_DEV) % N_DEV
        hop_l = (owner - my_pos - 1 + 2 * N_DEV) % N_DEV
        s_r = (stack_iota == hop_r * CAP + pos_own).astype(jnp.bfloat16)
        s_l = (stack_iota == hop_l * CAP + pos_own).astype(jnp.bfloat16)

        blk_r = blkr_ref[:, :, :].reshape(N_DEV * CAP, half)
        blk_l = blkl_ref[:, :, :].reshape(N_DEV * CAP, half)
        out_ref[:, :half] = part_ref[:, :half] + jnp.dot(
            s_r, blk_r, preferred_element_type=jnp.float32
        )
        out_ref[:, half:] = part_ref[:, half:] + jnp.dot(
            s_l, blk_l, preferred_element_type=jnp.float32
        )

    part, blkr, blkl, pos = pl.pallas_call(
        body_a,
        out_shape=[
            jax.ShapeDtypeStruct((n_tok, d_ff), jnp.float32),
            jax.ShapeDtypeStruct((N_DEV, CAP, half), jnp.bfloat16),
            jax.ShapeDtypeStruct((N_DEV, CAP, half), jnp.bfloat16),
            jax.ShapeDtypeStruct((n_tok, N_DEV), jnp.float32),
        ],
        in_specs=[pl.BlockSpec(memory_space=pltpu.VMEM)] * 5,
        out_specs=[pl.BlockSpec(memory_space=pltpu.VMEM)] * 4,
        scratch_shapes=[
            pltpu.VMEM((2, CAP, half), jnp.bfloat16),
            pltpu.VMEM((2, CAP, half), jnp.bfloat16),
            pltpu.SemaphoreType.DMA((N_DEV - 1,)),
            pltpu.SemaphoreType.DMA((N_DEV - 1,)),
            pltpu.SemaphoreType.DMA((N_DEV - 1,)),
            pltpu.SemaphoreType.DMA((N_DEV - 1,)),
        ],
        compiler_params=pltpu.CompilerParams(collective_id=0),
    )(x, router_W, route_idx, expert_W, shared_W)

    return pl.pallas_call(
        body_b,
        out_shape=jax.ShapeDtypeStruct((n_tok, d_ff), jnp.float32),
        in_specs=[pl.BlockSpec(memory_space=pltpu.VMEM)] * 5,
        out_specs=pl.BlockSpec(memory_space=pltpu.VMEM),
        input_output_aliases={2: 0},
    )(route_idx, pos, part, blkr, blkl)


# device time: 26064 ns/iter; 1.0985x vs baseline; 1.0985x over previous
import jax
import jax.numpy as jnp
from jax import lax
from jax.experimental import pallas as pl
from jax.experimental.pallas import tpu as pltpu

N_DEV = 4
CAP = 384


def kernel(x, router_W, route_idx, expert_W, shared_W):
    n_tok, d_model = x.shape
    e_local = expert_W.shape[0]
    d_ff = shared_W.shape[1]
    n_exp = router_W.shape[1]
    half = d_ff // 2

    def body_a(x_ref, rw_ref, idx_ref, ew_ref, sw_ref,
               part_ref, blkr_ref, blkl_ref, pos_ref,
               comm_r, comm_l, send_r, recv_r, send_l, recv_l):
        my_pos = lax.axis_index("i")
        left = (my_pos + N_DEV - 1) % N_DEV
        right = (my_pos + 1) % N_DEV

        barrier_sem = pltpu.get_barrier_semaphore()
        for nbr in (left, right):
            pl.semaphore_signal(
                barrier_sem, inc=1,
                device_id=(nbr,), device_id_type=pl.DeviceIdType.MESH,
            )
        pl.semaphore_wait(barrier_sem, 2)

        xf = x_ref[:, :]

        scores = jnp.dot(xf, rw_ref[:, :], preferred_element_type=jnp.float32)
        s_max = jnp.max(scores, axis=-1, keepdims=True)
        p = jnp.exp(scores - s_max)
        probs = p / jnp.sum(p, axis=-1, keepdims=True)

        route = idx_ref[:, :]
        eidx = lax.broadcasted_iota(jnp.int32, (n_tok, n_exp), 1)
        onehot = (eidx == route).astype(jnp.float32)
        p_tok = jnp.sum(probs * onehot, axis=-1, keepdims=True)

        route_chip = route // e_local
        cidx = lax.broadcasted_iota(jnp.int32, (n_tok, N_DEV), 1)
        chip_masks = (cidx == route_chip).astype(jnp.float32)
        r_iota = lax.broadcasted_iota(jnp.int32, (n_tok, n_tok), 0)
        c_iota = lax.broadcasted_iota(jnp.int32, (n_tok, n_tok), 1)
        l_strict = (c_iota < r_iota).astype(jnp.bfloat16)
        pos_f = jnp.dot(
            l_strict, chip_masks.astype(jnp.bfloat16),
            preferred_element_type=jnp.float32,
        )
        pos_ref[:, :] = pos_f

        slot_iota = lax.broadcasted_iota(jnp.int32, (n_tok, CAP), 1)
        mask_me = (route_chip == my_pos).astype(jnp.float32)
        pos_me = jnp.sum(
            pos_f * (cidx == my_pos).astype(jnp.float32),
            axis=1, keepdims=True,
        ).astype(jnp.int32)
        s_me = (
            (slot_iota == pos_me).astype(jnp.float32) * mask_me
        ).astype(jnp.bfloat16)

        x_scaled = (xf * p_tok).astype(jnp.bfloat16)
        x_c = lax.dot_general(
            s_me, x_scaled,
            dimension_numbers=(((0,), (0,)), ((), ())),
            preferred_element_type=jnp.float32,
        ).astype(jnp.bfloat16)
        le_iota = lax.broadcasted_iota(jnp.int32, (n_tok, e_local), 1)
        local_onehot = (
            le_iota == route - my_pos * e_local
        ).astype(jnp.bfloat16)
        masks_c = lax.dot_general(
            s_me, local_onehot,
            dimension_numbers=(((0,), (0,)), ((), ())),
            preferred_element_type=jnp.float32,
        ).astype(jnp.bfloat16)

        y_c = jnp.zeros((CAP, d_ff), jnp.float32)
        for e in range(e_local):
            xs = x_c * masks_c[:, e:e + 1]
            y_c = y_c + jnp.dot(
                xs, ew_ref[e, :, :].astype(jnp.bfloat16),
                preferred_element_type=jnp.float32,
            )
        y_bf = y_c.astype(jnp.bfloat16)

        comm_r[0, :, :] = y_bf[:, :half]
        comm_l[0, :, :] = y_bf[:, half:]
        blkr_ref[N_DEV - 1, :, :] = y_bf[:, :half]
        blkl_ref[N_DEV - 1, :, :] = y_bf[:, half:]

        chunk = CAP // 2

        def make_hop(h, c):
            send_slot = h % 2
            recv_slot = (h + 1) % 2
            rows = pl.ds(c * chunk, chunk)
            rdma_r = pltpu.make_async_remote_copy(
                src_ref=comm_r.at[send_slot, rows],
                dst_ref=comm_r.at[recv_slot, rows],
                send_sem=send_r.at[h, c],
                recv_sem=recv_r.at[h, c],
                device_id=(right,),
                device_id_type=pl.DeviceIdType.MESH,
            )
            rdma_l = pltpu.make_async_remote_copy(
                src_ref=comm_l.at[send_slot, rows],
                dst_ref=comm_l.at[recv_slot, rows],
                send_sem=send_l.at[h, c],
                recv_sem=recv_l.at[h, c],
                device_id=(left,),
                device_id_type=pl.DeviceIdType.MESH,
            )
            rdma_r.start()
            rdma_l.start()
            return rdma_r, rdma_l

        hop = [make_hop(0, 0), make_hop(0, 1)]
        part_ref[:, :] = jnp.dot(
            xf.astype(jnp.bfloat16), sw_ref[:, :].astype(jnp.bfloat16),
            preferred_element_type=jnp.float32,
        ).astype(jnp.bfloat16)

        for h in range(N_DEV - 1):
            nxt = [None, None]
            for c in range(2):
                hop[c][0].wait()
                hop[c][1].wait()
                recv_slot = (h + 1) % 2
                if h + 1 < N_DEV - 1:
                    nxt[c] = make_hop(h + 1, c)
                rows = pl.ds(c * chunk, chunk)
                blkr_ref[h, rows, :] = comm_r[recv_slot, rows, :]
                blkl_ref[h, rows, :] = comm_l[recv_slot, rows, :]
            hop = nxt

    def body_b(idx_ref, pos_ref, part_ref, blkr_ref, blkl_ref, out_ref):
        my_pos = lax.axis_index("i")

        route = idx_ref[:, :]
        owner = route // e_local
        cidx = lax.broadcasted_iota(jnp.int32, (n_tok, N_DEV), 1)
        pos_f = pos_ref[:, :]
        pos_own = jnp.sum(
            pos_f * (cidx == owner).astype(jnp.float32),
            axis=1, keepdims=True,
        ).astype(jnp.int32)

        stack_iota = lax.broadcasted_iota(
            jnp.int32, (n_tok, N_DEV * CAP), 1
        )
        hop_r = (my_pos - owner - 1 + 2 * N_DEV) % N_DEV
        hop_l = (owner - my_pos - 1 + 2 * N_DEV) % N_DEV
        s_r = (stack_iota == hop_r * CAP + pos_own).astype(jnp.bfloat16)
        s_l = (stack_iota == hop_l * CAP + pos_own).astype(jnp.bfloat16)

        blk_r = blkr_ref[:, :, :].reshape(N_DEV * CAP, half)
        blk_l = blkl_ref[:, :, :].reshape(N_DEV * CAP, half)
        part = part_ref[:, :].astype(jnp.float32)
        out_ref[:, :half] = part[:, :half] + jnp.dot(
            s_r, blk_r, preferred_element_type=jnp.float32
        )
        out_ref[:, half:] = part[:, half:] + jnp.dot(
            s_l, blk_l, preferred_element_type=jnp.float32
        )

    part, blkr, blkl, pos = pl.pallas_call(
        body_a,
        out_shape=[
            jax.ShapeDtypeStruct((n_tok, d_ff), jnp.bfloat16),
            jax.ShapeDtypeStruct((N_DEV, CAP, half), jnp.bfloat16),
            jax.ShapeDtypeStruct((N_DEV, CAP, half), jnp.bfloat16),
            jax.ShapeDtypeStruct((n_tok, N_DEV), jnp.float32),
        ],
        in_specs=[pl.BlockSpec(memory_space=pltpu.VMEM)] * 5,
        out_specs=[pl.BlockSpec(memory_space=pltpu.VMEM)] * 4,
        scratch_shapes=[
            pltpu.VMEM((2, CAP, half), jnp.bfloat16),
            pltpu.VMEM((2, CAP, half), jnp.bfloat16),
            pltpu.SemaphoreType.DMA((N_DEV - 1, 2)),
            pltpu.SemaphoreType.DMA((N_DEV - 1, 2)),
            pltpu.SemaphoreType.DMA((N_DEV - 1, 2)),
            pltpu.SemaphoreType.DMA((N_DEV - 1, 2)),
        ],
        compiler_params=pltpu.CompilerParams(collective_id=0),
    )(x, router_W, route_idx, expert_W, shared_W)

    return pl.pallas_call(
        body_b,
        out_shape=jax.ShapeDtypeStruct((n_tok, d_ff), jnp.float32),
        in_specs=[pl.BlockSpec(memory_space=pltpu.VMEM)] * 5,
        out_specs=pl.BlockSpec(memory_space=pltpu.VMEM),
    )(route_idx, pos, part, blkr, blkl)


# device time: 25234 ns/iter; 1.1346x vs baseline; 1.0329x over previous
import jax
import jax.numpy as jnp
from jax import lax
from jax.experimental import pallas as pl
from jax.experimental.pallas import tpu as pltpu

N_DEV = 4
CAP = 384


def kernel(x, router_W, route_idx, expert_W, shared_W):
    n_tok, d_model = x.shape
    e_local = expert_W.shape[0]
    d_ff = shared_W.shape[1]
    n_exp = router_W.shape[1]
    half = d_ff // 2

    def body_a(x_ref, rw_ref, idx_ref, ew_ref, sw_ref,
               part_ref, blkr_ref, blkl_ref, pos_ref,
               comm_r, comm_l, send_r, recv_r, send_l, recv_l):
        my_pos = lax.axis_index("i")
        left = (my_pos + N_DEV - 1) % N_DEV
        right = (my_pos + 1) % N_DEV

        barrier_sem = pltpu.get_barrier_semaphore()
        for nbr in (left, right):
            pl.semaphore_signal(
                barrier_sem, inc=1,
                device_id=(nbr,), device_id_type=pl.DeviceIdType.MESH,
            )
        pl.semaphore_wait(barrier_sem, 2)

        xf = x_ref[:, :]

        scores = jnp.dot(xf, rw_ref[:, :], preferred_element_type=jnp.float32)
        s_max = jnp.max(scores, axis=-1, keepdims=True)
        p = jnp.exp(scores - s_max)
        probs = p / jnp.sum(p, axis=-1, keepdims=True)

        route = idx_ref[:, :]
        eidx = lax.broadcasted_iota(jnp.int32, (n_tok, n_exp), 1)
        onehot = (eidx == route).astype(jnp.float32)
        p_tok = jnp.sum(probs * onehot, axis=-1, keepdims=True)

        route_chip = route // e_local
        cidx = lax.broadcasted_iota(jnp.int32, (n_tok, N_DEV), 1)
        chip_masks = (cidx == route_chip).astype(jnp.float32)
        r_iota = lax.broadcasted_iota(jnp.int32, (n_tok, n_tok), 0)
        c_iota = lax.broadcasted_iota(jnp.int32, (n_tok, n_tok), 1)
        l_strict = (c_iota < r_iota).astype(jnp.bfloat16)
        pos_f = jnp.dot(
            l_strict, chip_masks.astype(jnp.bfloat16),
            preferred_element_type=jnp.float32,
        )
        pos_ref[:, :] = pos_f

        slot_iota = lax.broadcasted_iota(jnp.int32, (n_tok, CAP), 1)
        mask_me = (route_chip == my_pos).astype(jnp.float32)
        pos_me = jnp.sum(
            pos_f * (cidx == my_pos).astype(jnp.float32),
            axis=1, keepdims=True,
        ).astype(jnp.int32)
        s_me = (
            (slot_iota == pos_me).astype(jnp.float32) * mask_me
        ).astype(jnp.bfloat16)

        x_scaled = (xf * p_tok).astype(jnp.bfloat16)
        x_c = lax.dot_general(
            s_me, x_scaled,
            dimension_numbers=(((0,), (0,)), ((), ())),
            preferred_element_type=jnp.float32,
        ).astype(jnp.bfloat16)
        le_iota = lax.broadcasted_iota(jnp.int32, (n_tok, e_local), 1)
        local_onehot = (
            le_iota == route - my_pos * e_local
        ).astype(jnp.bfloat16)
        masks_c = lax.dot_general(
            s_me, local_onehot,
            dimension_numbers=(((0,), (0,)), ((), ())),
            preferred_element_type=jnp.float32,
        ).astype(jnp.bfloat16)

        y_c = jnp.zeros((CAP, d_ff), jnp.float32)
        for e in range(e_local):
            xs = x_c * masks_c[:, e:e + 1]
            y_c = y_c + jnp.dot(
                xs, ew_ref[e, :, :].astype(jnp.bfloat16),
                preferred_element_type=jnp.float32,
            )
        y_bf = y_c.astype(jnp.bfloat16)

        comm_r[0, :, :] = y_bf[:, :half]
        comm_l[0, :, :] = y_bf[:, half:]
        blkr_ref[N_DEV - 1, :, :] = y_bf[:, :half]
        blkl_ref[N_DEV - 1, :, :] = y_bf[:, half:]

        n_chunks = 4
        chunk = CAP // n_chunks

        def make_hop(h, c):
            send_slot = h % 2
            recv_slot = (h + 1) % 2
            rows = pl.ds(c * chunk, chunk)
            rdma_r = pltpu.make_async_remote_copy(
                src_ref=comm_r.at[send_slot, rows],
                dst_ref=comm_r.at[recv_slot, rows],
                send_sem=send_r.at[h, c],
                recv_sem=recv_r.at[h, c],
                device_id=(right,),
                device_id_type=pl.DeviceIdType.MESH,
            )
            rdma_l = pltpu.make_async_remote_copy(
                src_ref=comm_l.at[send_slot, rows],
                dst_ref=comm_l.at[recv_slot, rows],
                send_sem=send_l.at[h, c],
                recv_sem=recv_l.at[h, c],
                device_id=(left,),
                device_id_type=pl.DeviceIdType.MESH,
            )
            rdma_r.start()
            rdma_l.start()
            return rdma_r, rdma_l

        hop = [make_hop(0, c) for c in range(n_chunks)]
        part_ref[:, :] = jnp.dot(
            xf.astype(jnp.bfloat16), sw_ref[:, :].astype(jnp.bfloat16),
            preferred_element_type=jnp.float32,
        ).astype(jnp.bfloat16)

        for h in range(N_DEV - 1):
            nxt = [None] * n_chunks
            for c in range(n_chunks):
                hop[c][0].wait()
                hop[c][1].wait()
                recv_slot = (h + 1) % 2
                if h + 1 < N_DEV - 1:
                    nxt[c] = make_hop(h + 1, c)
                rows = pl.ds(c * chunk, chunk)
                blkr_ref[h, rows, :] = comm_r[recv_slot, rows, :]
                blkl_ref[h, rows, :] = comm_l[recv_slot, rows, :]
            hop = nxt

    def body_b(idx_ref, pos_ref, part_ref, blkr_ref, blkl_ref, out_ref):
        my_pos = lax.axis_index("i")

        route = idx_ref[:, :]
        owner = route // e_local
        cidx = lax.broadcasted_iota(jnp.int32, (n_tok, N_DEV), 1)
        pos_f = pos_ref[:, :]
        pos_own = jnp.sum(
            pos_f * (cidx == owner).astype(jnp.float32),
            axis=1, keepdims=True,
        ).astype(jnp.int32)

        stack_iota = lax.broadcasted_iota(
            jnp.int32, (n_tok, N_DEV * CAP), 1
        )
        hop_r = (my_pos - owner - 1 + 2 * N_DEV) % N_DEV
        hop_l = (owner - my_pos - 1 + 2 * N_DEV) % N_DEV
        s_r = (stack_iota == hop_r * CAP + pos_own).astype(jnp.bfloat16)
        s_l = (stack_iota == hop_l * CAP + pos_own).astype(jnp.bfloat16)

        blk_r = blkr_ref[:, :, :].reshape(N_DEV * CAP, half)
        blk_l = blkl_ref[:, :, :].reshape(N_DEV * CAP, half)
        part = part_ref[:, :].astype(jnp.float32)
        out_ref[:, :half] = part[:, :half] + jnp.dot(
            s_r, blk_r, preferred_element_type=jnp.float32
        )
        out_ref[:, half:] = part[:, half:] + jnp.dot(
            s_l, blk_l, preferred_element_type=jnp.float32
        )

    part, blkr, blkl, pos = pl.pallas_call(
        body_a,
        out_shape=[
            jax.ShapeDtypeStruct((n_tok, d_ff), jnp.bfloat16),
            jax.ShapeDtypeStruct((N_DEV, CAP, half), jnp.bfloat16),
            jax.ShapeDtypeStruct((N_DEV, CAP, half), jnp.bfloat16),
            jax.ShapeDtypeStruct((n_tok, N_DEV), jnp.float32),
        ],
        in_specs=[pl.BlockSpec(memory_space=pltpu.VMEM)] * 5,
        out_specs=[pl.BlockSpec(memory_space=pltpu.VMEM)] * 4,
        scratch_shapes=[
            pltpu.VMEM((2, CAP, half), jnp.bfloat16),
            pltpu.VMEM((2, CAP, half), jnp.bfloat16),
            pltpu.SemaphoreType.DMA((N_DEV - 1, 4)),
            pltpu.SemaphoreType.DMA((N_DEV - 1, 4)),
            pltpu.SemaphoreType.DMA((N_DEV - 1, 4)),
            pltpu.SemaphoreType.DMA((N_DEV - 1, 4)),
        ],
        compiler_params=pltpu.CompilerParams(collective_id=0),
    )(x, router_W, route_idx, expert_W, shared_W)

    return pl.pallas_call(
        body_b,
        out_shape=jax.ShapeDtypeStruct((n_tok, d_ff), jnp.float32),
        in_specs=[pl.BlockSpec(memory_space=pltpu.VMEM)] * 5,
        out_specs=pl.BlockSpec(memory_space=pltpu.VMEM),
    )(route_idx, pos, part, blkr, blkl)
